# baseline (speedup 1.0000x reference)
import jax
import jax.numpy as jnp
from jax.experimental import pallas as pl
from jax.experimental.pallas import tpu as pltpu

B, NV, D, P = 64, 7, 768, 64
N = B * NV
K = D * P
OUTF = 96
E, RANK = 16, 8
ER = E * RANK
HID = 384
SCALING = 16.0 / RANK

_f32 = jnp.float32
_bf16 = jnp.bfloat16


def _stage1(x_ref, wb_ref, a_ref, base_ref, t_ref, ps_ref):
    p = pl.program_id(0)
    xp = x_ref[...].reshape(N, D)
    xp16 = xp.astype(_bf16)
    wb = wb_ref[...].reshape(OUTF, D)
    ab = a_ref[...].reshape(ER, D)
    dn = (((1,), (1,)), ((), ()))
    base_c = jax.lax.dot_general(xp16, wb, dn, preferred_element_type=_f32)
    t_c = jax.lax.dot_general(xp16, ab, dn, preferred_element_type=_f32)

    @pl.when(p == 0)
    def _():
        base_ref[...] = base_c
        t_ref[...] = t_c
        ps_ref[...] = xp

    @pl.when(p > 0)
    def _():
        base_ref[...] += base_c
        t_ref[...] += t_c
        ps_ref[...] += xp


def _stage2(base_ref, t_ref, ps_ref, w1_ref, b1_ref, w2_ref, b2_ref,
            bb_ref, bigb_ref, out_ref, probs_ref):
    hi = jax.lax.Precision.HIGHEST
    dnT = (((1,), (1,)), ((), ()))

    gv_r = jax.lax.broadcasted_iota(jnp.int32, (B, N), 0)
    gv_c = jax.lax.broadcasted_iota(jnp.int32, (B, N), 1)
    gv = (gv_c // NV == gv_r).astype(_f32)
    pooled = jax.lax.dot_general(
        gv, ps_ref[...], (((1,), (0,)), ((), ())),
        preferred_element_type=_f32, precision=hi) * (1.0 / (NV * P))

    h = jax.lax.dot_general(pooled, w1_ref[...], dnT,
                            preferred_element_type=_f32, precision=hi)
    h = h + b1_ref[...]
    h = 0.5 * h * (1.0 + jax.lax.erf(h * 0.7071067811865476))
    logits = jax.lax.dot_general(h, w2_ref[...], dnT,
                                 preferred_element_type=_f32, precision=hi)
    logits = logits + b2_ref[...]
    m = jnp.max(logits, axis=-1, keepdims=True)
    ex = jnp.exp(logits - m)
    probs = ex / jnp.sum(ex, axis=-1, keepdims=True)
    probs_ref[...] = probs

    lane = jax.lax.broadcasted_iota(jnp.int32, (B, E), 1)
    i1 = jnp.argmax(probs, axis=-1)[:, None]
    oh1 = (lane == i1)
    w1v = jnp.max(probs, axis=-1, keepdims=True)
    masked = jnp.where(oh1, -1.0, probs)
    i2 = jnp.argmax(masked, axis=-1)[:, None]
    oh2 = (lane == i2)
    w2v = jnp.max(masked, axis=-1, keepdims=True)
    denom = jnp.maximum(w1v + w2v, 1e-6)
    wfull = (oh1.astype(_f32) * w1v + oh2.astype(_f32) * w2v) / denom

    r_r = jax.lax.broadcasted_iota(jnp.int32, (E, ER), 0)
    r_c = jax.lax.broadcasted_iota(jnp.int32, (E, ER), 1)
    rmat = (r_c // RANK == r_r).astype(_f32)
    wbig = jax.lax.dot_general(wfull, rmat, (((1,), (0,)), ((), ())),
                               preferred_element_type=_f32, precision=hi)
    gt_r = jax.lax.broadcasted_iota(jnp.int32, (N, B), 0)
    gt_c = jax.lax.broadcasted_iota(jnp.int32, (N, B), 1)
    gvt = (gt_r // NV == gt_c).astype(_f32)
    vbig = jax.lax.dot_general(gvt, wbig, (((1,), (0,)), ((), ())),
                               preferred_element_type=_f32, precision=hi)

    tw = t_ref[...] * vbig
    moe = jax.lax.dot_general(tw, bigb_ref[...], (((1,), (0,)), ((), ())),
                              preferred_element_type=_f32, precision=hi)
    out_ref[...] = base_ref[...] + bb_ref[...] + moe


def kernel(x, W_base, b_base, W1, b1, W2, b2, lora_A, lora_B):
    xT = jnp.transpose(x, (0, 1, 3, 2)).reshape(N, P, 1, D)
    wt = jnp.transpose(W_base.reshape(OUTF, D, P).astype(_bf16), (2, 0, 1))
    at = jnp.transpose(lora_A.reshape(ER, D, P).astype(_bf16), (2, 0, 1))

    base_acc, t_acc, ps = pl.pallas_call(
        _stage1,
        grid=(P,),
        in_specs=[
            pl.BlockSpec((N, 1, 1, D), lambda p: (0, p, 0, 0)),
            pl.BlockSpec((1, OUTF, D), lambda p: (p, 0, 0)),
            pl.BlockSpec((1, ER, D), lambda p: (p, 0, 0)),
        ],
        out_specs=[
            pl.BlockSpec((N, OUTF), lambda p: (0, 0)),
            pl.BlockSpec((N, ER), lambda p: (0, 0)),
            pl.BlockSpec((N, D), lambda p: (0, 0)),
        ],
        out_shape=[
            jax.ShapeDtypeStruct((N, OUTF), _f32),
            jax.ShapeDtypeStruct((N, ER), _f32),
            jax.ShapeDtypeStruct((N, D), _f32),
        ],
    )(xT, wt, at)

    bigb = jnp.transpose(lora_B, (0, 2, 1)).reshape(ER, OUTF) * SCALING

    final, probs = pl.pallas_call(
        _stage2,
        out_shape=[
            jax.ShapeDtypeStruct((N, OUTF), _f32),
            jax.ShapeDtypeStruct((B, E), _f32),
        ],
    )(base_acc, t_acc, ps, W1, b1.reshape(1, HID), W2, b2.reshape(1, E),
      b_base.reshape(1, OUTF), bigb)

    return final.reshape(B, NV, OUTF), probs


# speedup vs baseline: 2.1400x; 2.1400x over previous
import jax
import jax.numpy as jnp
from jax.experimental import pallas as pl
from jax.experimental.pallas import tpu as pltpu

B, NV, D, P = 64, 7, 768, 64
N = B * NV
K = D * P
OUTF = 96
E, RANK = 16, 8
ER = E * RANK
HID = 384
SCALING = 16.0 / RANK

_f32 = jnp.float32
_bf16 = jnp.bfloat16


PB = 8
NPB = P // PB


def _stage1(x_ref, w_ref, bt_ref, ps_ref):
    k = pl.program_id(0)
    xb = x_ref[...]
    xb16 = xb.astype(_bf16)
    dn = (((1,), (1,)), ((), ()))
    bt = jax.lax.dot_general(xb16[:, 0, :], w_ref[0], dn,
                             preferred_element_type=_f32)
    ps = xb[:, 0, :]
    for i in range(1, PB):
        bt += jax.lax.dot_general(xb16[:, i, :], w_ref[i], dn,
                                  preferred_element_type=_f32)
        ps += xb[:, i, :]

    @pl.when(k == 0)
    def _():
        bt_ref[...] = bt
        ps_ref[...] = ps

    @pl.when(k > 0)
    def _():
        bt_ref[...] += bt
        ps_ref[...] += ps


def _stage2(bt_ref, ps_ref, w1_ref, b1_ref, w2_ref, b2_ref,
            bb_ref, bigb_ref, out_ref, probs_ref):
    bt = bt_ref[...]
    base = bt[:, :OUTF]
    t = bt[:, OUTF:]
    hi = jax.lax.Precision.HIGHEST
    dnT = (((1,), (1,)), ((), ()))

    gv_r = jax.lax.broadcasted_iota(jnp.int32, (B, N), 0)
    gv_c = jax.lax.broadcasted_iota(jnp.int32, (B, N), 1)
    gv = (gv_c // NV == gv_r).astype(_f32)
    pooled = jax.lax.dot_general(
        gv, ps_ref[...], (((1,), (0,)), ((), ())),
        preferred_element_type=_f32, precision=hi) * (1.0 / (NV * P))

    h = jax.lax.dot_general(pooled, w1_ref[...], dnT,
                            preferred_element_type=_f32, precision=hi)
    h = h + b1_ref[...]
    h = 0.5 * h * (1.0 + jax.lax.erf(h * 0.7071067811865476))
    logits = jax.lax.dot_general(h, w2_ref[...], dnT,
                                 preferred_element_type=_f32, precision=hi)
    logits = logits + b2_ref[...]
    m = jnp.max(logits, axis=-1, keepdims=True)
    ex = jnp.exp(logits - m)
    probs = ex / jnp.sum(ex, axis=-1, keepdims=True)
    probs_ref[...] = probs

    lane = jax.lax.broadcasted_iota(jnp.int32, (B, E), 1)
    i1 = jnp.argmax(probs, axis=-1)[:, None]
    oh1 = (lane == i1)
    w1v = jnp.max(probs, axis=-1, keepdims=True)
    masked = jnp.where(oh1, -1.0, probs)
    i2 = jnp.argmax(masked, axis=-1)[:, None]
    oh2 = (lane == i2)
    w2v = jnp.max(masked, axis=-1, keepdims=True)
    denom = jnp.maximum(w1v + w2v, 1e-6)
    wfull = (oh1.astype(_f32) * w1v + oh2.astype(_f32) * w2v) / denom

    r_r = jax.lax.broadcasted_iota(jnp.int32, (E, ER), 0)
    r_c = jax.lax.broadcasted_iota(jnp.int32, (E, ER), 1)
    rmat = (r_c // RANK == r_r).astype(_f32)
    wbig = jax.lax.dot_general(wfull, rmat, (((1,), (0,)), ((), ())),
                               preferred_element_type=_f32, precision=hi)
    gt_r = jax.lax.broadcasted_iota(jnp.int32, (N, B), 0)
    gt_c = jax.lax.broadcasted_iota(jnp.int32, (N, B), 1)
    gvt = (gt_r // NV == gt_c).astype(_f32)
    vbig = jax.lax.dot_general(gvt, wbig, (((1,), (0,)), ((), ())),
                               preferred_element_type=_f32, precision=hi)

    tw = t * vbig
    moe = jax.lax.dot_general(tw, bigb_ref[...], (((1,), (0,)), ((), ())),
                              preferred_element_type=_f32, precision=hi)
    out_ref[...] = base + bb_ref[...] + moe


def kernel(x, W_base, b_base, W1, b1, W2, b2, lora_A, lora_B):
    xT = jnp.transpose(x, (0, 1, 3, 2)).reshape(N, P, D)
    wt = jnp.transpose(W_base.reshape(OUTF, D, P).astype(_bf16), (2, 0, 1))
    at = jnp.transpose(lora_A.reshape(ER, D, P).astype(_bf16), (2, 0, 1))
    wcat = jnp.concatenate([wt, at], axis=1)

    bt_acc, ps = pl.pallas_call(
        _stage1,
        grid=(NPB,),
        in_specs=[
            pl.BlockSpec((N, PB, D), lambda k: (0, k, 0)),
            pl.BlockSpec((PB, OUTF + ER, D), lambda k: (k, 0, 0)),
        ],
        out_specs=[
            pl.BlockSpec((N, OUTF + ER), lambda k: (0, 0)),
            pl.BlockSpec((N, D), lambda k: (0, 0)),
        ],
        out_shape=[
            jax.ShapeDtypeStruct((N, OUTF + ER), _f32),
            jax.ShapeDtypeStruct((N, D), _f32),
        ],
    )(xT, wcat)

    bigb = jnp.transpose(lora_B, (0, 2, 1)).reshape(ER, OUTF) * SCALING

    final, probs = pl.pallas_call(
        _stage2,
        out_shape=[
            jax.ShapeDtypeStruct((N, OUTF), _f32),
            jax.ShapeDtypeStruct((B, E), _f32),
        ],
    )(bt_acc, ps, W1, b1.reshape(1, HID), W2, b2.reshape(1, E),
      b_base.reshape(1, OUTF), bigb)

    return final.reshape(B, NV, OUTF), probs
